# baseline (speedup 1.0000x reference)
import functools

import jax
import jax.numpy as jnp
from jax import lax
from jax.experimental import pallas as pl
from jax.experimental.pallas import tpu as pltpu
from jax.experimental.pallas import tpu_sc as plsc

_B, _F, _K, _D = 16384, 100, 50, 64
_NW = 32
_ROWS_PER_W = _B // _NW
_CHUNK = 32
_NCHUNK = _ROWS_PER_W // _CHUNK


def _make_sc_kernel():
    mesh = plsc.VectorSubcoreMesh(core_axis_name="c", subcore_axis_name="s")

    @functools.partial(
        pl.kernel,
        mesh=mesh,
        out_type=jax.ShapeDtypeStruct((_B, _K, _D), jnp.float32),
        scratch_types=[
            pltpu.VMEM((_CHUNK, _K, _D), jnp.float32),
        ],
        compiler_params=pltpu.CompilerParams(use_tc_tiling_on_sc=False),
    )
    def sc_copy(in_hbm, out_hbm, buf):
        wid = lax.axis_index("s") * 2 + lax.axis_index("c")
        base = wid * _ROWS_PER_W

        def body(g, carry):
            b0 = base + g * _CHUNK
            pltpu.sync_copy(in_hbm.at[pl.ds(b0, _CHUNK), :, pl.ds(0, _D)], buf)
            pltpu.sync_copy(buf, out_hbm.at[pl.ds(b0, _CHUNK)])
            return carry

        lax.fori_loop(0, _NCHUNK, body, 0)

    return sc_copy


_SC_KERNEL = _make_sc_kernel()


def kernel(tensor):
    in3 = tensor.reshape(_B, _K, 2 * _D)
    return _SC_KERNEL(in3)
